# baseline (speedup 1.0000x reference)
import functools

import jax
import jax.numpy as jnp
from jax import lax
from jax.experimental import pallas as pl
from jax.experimental.pallas import tpu as pltpu
from jax.experimental.pallas import tpu_sc as plsc

_B = 4096
_T = 200
_V = 1000000
_D = 16
_BOS = 2

_NC = 2
_NS = 16
_NW = _NC * _NS
_ROWS_W = _B * _T // _NW
_CH = 128
_NCH = _ROWS_W // _CH
_SEQ_W = _ROWS_W // _T
_R = 6
_G = 3


def _body(inp_ref, tab_ref, out_ref, idx_v, rows_v, bos_rows, bos_idx, pos_v,
          gsem, wsem, sem0):
    wid = lax.axis_index("s") * _NC + lax.axis_index("c")
    base = wid * _ROWS_W

    pltpu.sync_copy(inp_ref.at[wid], idx_v)

    lanes = lax.iota(jnp.int32, 16)
    for i in range(_SEQ_W // 16):
        bos_idx[pl.ds(i * 16, 16)] = jnp.full((16,), _BOS, jnp.int32)
        pos_v[pl.ds(i * 16, 16)] = (lanes + i * 16) * _T + base

    pltpu.async_copy(tab_ref.at[bos_idx], bos_rows, sem0).wait()

    def _gather(c, s):
        return pltpu.make_async_copy(
            tab_ref.at[idx_v.at[c]], rows_v.at[s], gsem.at[s])

    def _write(c, s):
        return pltpu.make_async_copy(
            rows_v.at[s], out_ref.at[pl.ds(base + 1 + _CH * c, _CH)],
            wsem.at[s])

    def _step(c, carry):
        s = lax.rem(c, _R)

        @pl.when(c >= _R)
        def _wait_slot():
            _write(c - _R, s).wait()

        _gather(c, s).start()

        @pl.when(c >= _G)
        def _drain():
            s2 = lax.rem(c - _G, _R)
            _gather(c - _G, s2).wait()
            _write(c - _G, s2).start()

        return carry

    lax.fori_loop(0, _NCH, _step, None)

    for c in range(_NCH - _G, _NCH):
        s = c % _R
        _gather(c, s).wait()
        if c == _NCH - 1:
            pltpu.async_copy(
                rows_v.at[s, pl.ds(0, _CH - 1)],
                out_ref.at[pl.ds(base + 1 + _CH * c, _CH - 1)],
                wsem.at[s]).wait()
        else:
            _write(c, s).start()
    for c in range(_NCH - _R, _NCH - 1):
        _write(c, c % _R).wait()

    pltpu.async_copy(bos_rows, out_ref.at[pos_v], sem0).wait()


@jax.jit
def kernel(inputs, vocab_table):
    inputs3d = inputs.reshape(_NW, _NCH, _CH)
    call = pl.kernel(
        _body,
        out_type=jax.ShapeDtypeStruct((_B * _T, _D), jnp.float32),
        mesh=plsc.VectorSubcoreMesh(core_axis_name="c", subcore_axis_name="s"),
        compiler_params=pltpu.CompilerParams(use_tc_tiling_on_sc=False),
        scratch_types=[
            pltpu.VMEM((_NCH, _CH), jnp.int32),
            pltpu.VMEM((_R, _CH, _D), jnp.float32),
            pltpu.VMEM((_SEQ_W, _D), jnp.float32),
            pltpu.VMEM((_SEQ_W,), jnp.int32),
            pltpu.VMEM((_SEQ_W,), jnp.int32),
            pltpu.SemaphoreType.DMA((_R,)),
            pltpu.SemaphoreType.DMA((_R,)),
            pltpu.SemaphoreType.DMA,
        ],
    )
    out = call(inputs3d, vocab_table)
    return out.reshape(_B, _T, _D)
